# baseline (speedup 1.0000x reference)
import functools

import jax
import jax.numpy as jnp
from jax import lax
from jax.experimental import pallas as pl
from jax.experimental.pallas import tpu as pltpu
from jax.experimental.pallas import tpu_sc as plsc

PAD = 1
LN_EPS = 1e-12
H = 768
L = 16
NH = H // L
NW = 32
R = 64


def _pid_body(ids_ref, out_ref):
    ids = ids_ref[...]
    m = (ids != PAD).astype(jnp.int32)
    s = ids.shape[1]
    acc = m
    k = 1
    while k < s:
        z = jnp.zeros((ids.shape[0], k), jnp.int32)
        acc = acc + jnp.concatenate([z, acc[:, : s - k]], axis=1)
        k *= 2
    out_ref[...] = acc * m + PAD


def _position_ids(ids32):
    return pl.pallas_call(
        _pid_body,
        out_shape=jax.ShapeDtypeStruct(ids32.shape, jnp.int32),
    )(ids32)


_DNUMS = lax.GatherDimensionNumbers(
    offset_dims=(), collapsed_slice_dims=(0,), start_index_map=(0,))


def _allreduce_sum(x):
    lanes = lax.iota(jnp.int32, L)
    for k in (1, 2, 4, 8):
        idx = lax.bitwise_xor(lanes, jnp.int32(k))
        sh = lax.gather(x, idx[:, None], _DNUMS, slice_sizes=(1,),
                        mode=lax.GatherScatterMode.PROMISE_IN_BOUNDS)
        x = x + sh
    return x


def _sc_body(word_hbm, pos_hbm, type_hbm, gamma_hbm, beta_hbm, idw_hbm,
             idp_hbm, out_hbm, ty_v, gamma_v, beta_v, idw_v, idp_v, w_v, p_v,
             sem_w, sem_p):
    wid = lax.axis_index("s") * 2 + lax.axis_index("c")
    rows_per_w = out_hbm.shape[0] // NW
    base = wid * rows_per_w
    pltpu.sync_copy(type_hbm.at[pl.ds(0, 1)], ty_v)
    pltpu.sync_copy(gamma_hbm, gamma_v)
    pltpu.sync_copy(beta_hbm, beta_v)

    for c in range(rows_per_w // R):
        cb = base + c * R
        pltpu.sync_copy(idw_hbm.at[pl.ds(cb, R)], idw_v)
        pltpu.sync_copy(idp_hbm.at[pl.ds(cb, R)], idp_v)
        cw = pltpu.async_copy(word_hbm.at[idw_v], w_v, sem_w)
        cp = pltpu.async_copy(pos_hbm.at[idp_v], p_v, sem_p)
        cw.wait()
        cp.wait()

        def row(r, carry):
            sv = jnp.zeros((L,), jnp.float32)
            qv = jnp.zeros((L,), jnp.float32)
            for j in range(NH):
                sl = pl.ds(j * L, L)
                t = w_v[r, sl] + p_v[r, sl] + ty_v[0, sl]
                sv = sv + t
                qv = qv + t * t
                w_v[r, sl] = t
            meanv = _allreduce_sum(sv) * (1.0 / H)
            x = _allreduce_sum(qv) * (1.0 / H) - meanv * meanv + LN_EPS
            bits = lax.bitcast_convert_type(x, jnp.int32)
            y = lax.bitcast_convert_type(
                jnp.int32(0x5F3759DF) - lax.shift_right_logical(bits, 1),
                jnp.float32)
            for _ in range(4):
                y = y * (1.5 - 0.5 * x * y * y)
            for j in range(NH):
                sl = pl.ds(j * L, L)
                t = w_v[r, sl]
                w_v[r, sl] = (t - meanv) * y * gamma_v[sl] + beta_v[sl]
            return carry

        lax.fori_loop(0, R, row, 0)
        pltpu.sync_copy(w_v, out_hbm.at[pl.ds(cb, R)])


def _sc_call(word_emb, pos_emb, type_emb, ln_gamma, ln_beta, idw, idp):
    n = idw.shape[0]
    mesh = plsc.VectorSubcoreMesh(core_axis_name="c", subcore_axis_name="s")
    f = functools.partial(
        pl.kernel,
        mesh=mesh,
        out_type=jax.ShapeDtypeStruct((n, H), jnp.float32),
        scratch_types=[
            pltpu.VMEM((1, H), jnp.float32),
            pltpu.VMEM((H,), jnp.float32),
            pltpu.VMEM((H,), jnp.float32),
            pltpu.VMEM((R,), jnp.int32),
            pltpu.VMEM((R,), jnp.int32),
            pltpu.VMEM((R, H), jnp.float32),
            pltpu.VMEM((R, H), jnp.float32),
            pltpu.SemaphoreType.DMA,
            pltpu.SemaphoreType.DMA,
        ],
    )(_sc_body)
    return f(word_emb, pos_emb, type_emb, ln_gamma, ln_beta, idw, idp)


def kernel(input_ids, word_emb, type_emb, pos_emb, ln_gamma, ln_beta):
    b, s = input_ids.shape
    ids32 = input_ids.astype(jnp.int32)
    pid = _position_ids(ids32)
    out = _sc_call(word_emb, pos_emb, type_emb, ln_gamma, ln_beta,
                   ids32.reshape(-1), pid.reshape(-1))
    return out.reshape(b, s, H)


# speedup vs baseline: 1.2083x; 1.2083x over previous
import functools

import jax
import jax.numpy as jnp
from jax import lax
from jax.experimental import pallas as pl
from jax.experimental.pallas import tpu as pltpu
from jax.experimental.pallas import tpu_sc as plsc

PAD = 1
LN_EPS = 1e-12
H = 768
L = 16
NH = H // L
NW = 32
R = 32
NBUF = 2


def _pid_body(ids_ref, out_ref):
    ids = ids_ref[...]
    m = (ids != PAD).astype(jnp.int32)
    s = ids.shape[1]
    acc = m
    k = 1
    while k < s:
        z = jnp.zeros((ids.shape[0], k), jnp.int32)
        acc = acc + jnp.concatenate([z, acc[:, : s - k]], axis=1)
        k *= 2
    out_ref[...] = acc * m + PAD


def _position_ids(ids32):
    return pl.pallas_call(
        _pid_body,
        out_shape=jax.ShapeDtypeStruct(ids32.shape, jnp.int32),
    )(ids32)


def _fold_body(pos_ref, type_ref, out_ref):
    out_ref[...] = pos_ref[...] + type_ref[...][0:1, :]


def _fold_type(pos_emb, type_emb):
    n = pos_emb.shape[0]
    blk = 1024
    return pl.pallas_call(
        _fold_body,
        grid=(n // blk,),
        in_specs=[
            pl.BlockSpec((blk, H), lambda i: (i, 0)),
            pl.BlockSpec((2, H), lambda i: (0, 0)),
        ],
        out_specs=pl.BlockSpec((blk, H), lambda i: (i, 0)),
        out_shape=jax.ShapeDtypeStruct((n, H), jnp.float32),
    )(pos_emb, type_emb)


_DNUMS = lax.GatherDimensionNumbers(
    offset_dims=(), collapsed_slice_dims=(0,), start_index_map=(0,))


def _allreduce_sum(x):
    lanes = lax.iota(jnp.int32, L)
    for k in (1, 2, 4, 8):
        idx = lax.bitwise_xor(lanes, jnp.int32(k))
        sh = lax.gather(x, idx[:, None], _DNUMS, slice_sizes=(1,),
                        mode=lax.GatherScatterMode.PROMISE_IN_BOUNDS)
        x = x + sh
    return x


def _rsqrt(x):
    bits = lax.bitcast_convert_type(x, jnp.int32)
    y = lax.bitcast_convert_type(
        jnp.int32(0x5F3759DF) - lax.shift_right_logical(bits, 1), jnp.float32)
    for _ in range(3):
        y = y * (1.5 - 0.5 * x * y * y)
    return y


def _sc_body(word_hbm, pos_hbm, gamma_hbm, beta_hbm, idw_hbm,
             idp_hbm, out_hbm, gamma_v, beta_v, idw_v, idp_v, w_v, p_v,
             sem_w, sem_p):
    wid = lax.axis_index("s") * 2 + lax.axis_index("c")
    rows_per_w = out_hbm.shape[0] // NW
    nchunk = rows_per_w // R
    base = wid * rows_per_w
    pltpu.sync_copy(gamma_hbm, gamma_v)
    pltpu.sync_copy(beta_hbm, beta_v)

    def start(c, b):
        cb = base + c * R
        pltpu.sync_copy(idw_hbm.at[pl.ds(cb, R)], idw_v.at[b])
        pltpu.sync_copy(idp_hbm.at[pl.ds(cb, R)], idp_v.at[b])
        return (pltpu.async_copy(word_hbm.at[idw_v.at[b]], w_v.at[b], sem_w),
                pltpu.async_copy(pos_hbm.at[idp_v.at[b]], p_v.at[b], sem_p))

    def compute(c, b):
        cb = base + c * R

        def row(r, carry):
            sv = jnp.zeros((L,), jnp.float32)
            qv = jnp.zeros((L,), jnp.float32)
            for j in range(NH):
                sl = pl.ds(j * L, L)
                t = w_v[b, r, sl] + p_v[b, r, sl]
                sv = sv + t
                qv = qv + t * t
                w_v[b, r, sl] = t
            meanv = _allreduce_sum(sv) * (1.0 / H)
            var = _allreduce_sum(qv) * (1.0 / H) - meanv * meanv + LN_EPS
            y = _rsqrt(var)
            for j in range(NH):
                sl = pl.ds(j * L, L)
                t = w_v[b, r, sl]
                w_v[b, r, sl] = (t - meanv) * y * gamma_v[sl] + beta_v[sl]
            return carry

        lax.fori_loop(0, R, row, 0)
        pltpu.sync_copy(w_v.at[b], out_hbm.at[pl.ds(cb, R)])

    def wait_bufs(b):
        pltpu.make_async_copy(word_hbm.at[idw_v.at[b]], w_v.at[b],
                              sem_w).wait()
        pltpu.make_async_copy(pos_hbm.at[idp_v.at[b]], p_v.at[b],
                              sem_p).wait()

    start(0, 0)

    def pair(k, carry):
        c0 = k * 2
        start(c0 + 1, 1)
        wait_bufs(0)
        compute(c0, 0)
        start(jnp.minimum(c0 + 2, nchunk - 2), 0)
        wait_bufs(1)
        compute(c0 + 1, 1)
        return carry

    lax.fori_loop(0, nchunk // 2, pair, 0)
    wait_bufs(0)


def _sc_call(word_emb, pos2, ln_gamma, ln_beta, idw, idp):
    n = idw.shape[0]
    mesh = plsc.VectorSubcoreMesh(core_axis_name="c", subcore_axis_name="s")
    f = functools.partial(
        pl.kernel,
        mesh=mesh,
        out_type=jax.ShapeDtypeStruct((n, H), jnp.float32),
        scratch_types=[
            pltpu.VMEM((H,), jnp.float32),
            pltpu.VMEM((H,), jnp.float32),
            pltpu.VMEM((NBUF, R), jnp.int32),
            pltpu.VMEM((NBUF, R), jnp.int32),
            pltpu.VMEM((NBUF, R, H), jnp.float32),
            pltpu.VMEM((NBUF, R, H), jnp.float32),
            pltpu.SemaphoreType.DMA,
            pltpu.SemaphoreType.DMA,
        ],
    )(_sc_body)
    return f(word_emb, pos2, ln_gamma, ln_beta, idw, idp)


def kernel(input_ids, word_emb, type_emb, pos_emb, ln_gamma, ln_beta):
    b, s = input_ids.shape
    ids32 = input_ids.astype(jnp.int32)
    pid = _position_ids(ids32)
    pos2 = _fold_type(pos_emb, type_emb)
    out = _sc_call(word_emb, pos2, ln_gamma, ln_beta,
                   ids32.reshape(-1), pid.reshape(-1))
    return out.reshape(b, s, H)
